# Initial kernel scaffold; baseline (speedup 1.0000x reference)
import jax
import jax.numpy as jnp
from jax.experimental import pallas as pl


def kernel(x, img_size, W_conv1, b_conv1, W_score, b_score, W_loc, b_loc):
    raise NotImplementedError("write your pallas kernel here")


# speedup vs baseline: 9.2758x; 9.2758x over previous
import functools

import numpy as np
import jax
import jax.numpy as jnp
from jax import lax
from jax.experimental import pallas as pl
from jax.experimental.pallas import tpu as pltpu
from jax.experimental.pallas import tpu_sc as plsc

_RATIOS = (0.5, 1.0, 2.0)
_SCALES = (8, 16, 32)
_FEAT_STRIDE = 16
_BASE_SIZE = 16
_NMS_IOU = 0.7
_PRE_NMS = 3000
_POST_NMS = 300
_MIN_SIZE = 16.0

_H = 32
_W = 32
_P = _H * _W
_A = 9
_N = _P * _A
_NS = 3072
_B = 128
_NB = _NS // _B
_SEL = 384

_NEG = -1.0e38


def _anchor_grid():
    ab = np.zeros((9, 4), dtype=np.float32)
    for i, r in enumerate(_RATIOS):
        for j, s in enumerate(_SCALES):
            h = _BASE_SIZE * s * np.sqrt(r)
            w = _BASE_SIZE * s * np.sqrt(1.0 / r)
            k = i * 3 + j
            ab[k, 0] = -w / 2.0
            ab[k, 1] = -h / 2.0
            ab[k, 2] = w / 2.0
            ab[k, 3] = h / 2.0
    sx = np.arange(0, _W * _FEAT_STRIDE, _FEAT_STRIDE)
    sy = np.arange(0, _H * _FEAT_STRIDE, _FEAT_STRIDE)
    mx, my = np.meshgrid(sx, sy)
    shift = np.stack([mx.ravel(), my.ravel(), mx.ravel(), my.ravel()],
                     axis=1).astype(np.float32)
    return (ab[None, :, :] + shift[:, None, :]).reshape(-1, 4)


def _dot(a, b, prec):
    return lax.dot_general(a, b, (((1,), (0,)), ((), ())), precision=prec,
                           preferred_element_type=jnp.float32)


def _to_col(v_row, eye):
    return lax.dot_general(eye, v_row, (((1,), (1,)), ((), ())),
                           precision=lax.Precision.HIGHEST,
                           preferred_element_type=jnp.float32)


def _conv_kernel(x_ref, w_ref, out_ref):
    k = pl.program_id(0)
    part = _dot(x_ref[0], w_ref[0], lax.Precision.DEFAULT)

    @pl.when(k == 0)
    def _():
        out_ref[...] = part

    @pl.when(k > 0)
    def _():
        out_ref[...] = out_ref[...] + part


def _conv_trunk(xcol, w9):
    return pl.pallas_call(
        _conv_kernel,
        grid=(9,),
        in_specs=[
            pl.BlockSpec((1, _P, 512), lambda k: (k, 0, 0)),
            pl.BlockSpec((1, 512, 512), lambda k: (k, 0, 0)),
        ],
        out_specs=pl.BlockSpec((_P, 512), lambda k: (0, 0)),
        out_shape=jax.ShapeDtypeStruct((_P, 512), jnp.float32),
    )(xcol, w9)


def _head_kernel(f_ref, bc_ref, wh_ref, bh_ref, out_ref):
    f = jnp.maximum(f_ref[...] + bc_ref[...], 0.0)
    out_ref[...] = _dot(f, wh_ref[...], lax.Precision.DEFAULT) + bh_ref[...]


def _head(facc, bc, wh, bh):
    return pl.pallas_call(
        _head_kernel,
        out_shape=jax.ShapeDtypeStruct((_P, 128), jnp.float32),
    )(facc, bc, wh, bh)


def _decode_kernel(p_ref, img_ref, out_ref):
    dx, dy, dw, dh = p_ref[0], p_ref[1], p_ref[2], p_ref[3]
    s0, s1 = p_ref[4], p_ref[5]
    ax1, ay1, ax2, ay2 = p_ref[6], p_ref[7], p_ref[8], p_ref[9]
    imh = img_ref[0]
    imw = img_ref[1]

    sw = ax2 - ax1
    sh = ay2 - ay1
    cx0 = ax1 + 0.5 * sw
    cy0 = ay1 + 0.5 * sh
    cx = dx * sw + cx0
    cy = dy * sh + cy0
    ww = jnp.exp(dw) * sw
    hh = jnp.exp(dh) * sh
    x1 = jnp.clip(cx - 0.5 * ww, 0.0, imw)
    y1 = jnp.clip(cy - 0.5 * hh, 0.0, imh)
    x2 = jnp.clip(cx + 0.5 * ww, 0.0, imw)
    y2 = jnp.clip(cy + 0.5 * hh, 0.0, imh)
    valid = (x2 - x1 >= _MIN_SIZE) & (y2 - y1 >= _MIN_SIZE)

    m = jnp.maximum(s0, s1)
    e0 = jnp.exp(s0 - m)
    e1 = jnp.exp(s1 - m)
    fg = e1 / (e0 + e1)
    ms = jnp.where(valid, fg, _NEG)

    out_ref[0] = x1
    out_ref[1] = y1
    out_ref[2] = x2
    out_ref[3] = y2
    out_ref[4] = ms


def _decode(planes10, img_f):
    return pl.pallas_call(
        _decode_kernel,
        in_specs=[
            pl.BlockSpec(memory_space=pltpu.VMEM),
            pl.BlockSpec(memory_space=pltpu.SMEM),
        ],
        out_shape=jax.ShapeDtypeStruct((5, 72, 128), jnp.float32),
    )(planes10, img_f)


def _rank_kernel(ms_ref, out_ref):
    i = pl.program_id(0)
    n_chunk = _N // 1024
    a_row = ms_ref[pl.ds(i, 1)].reshape(1, 1024)
    i_idx = i * 1024 + lax.broadcasted_iota(jnp.int32, (1024, 1024), 1)
    eye = jnp.where(
        lax.broadcasted_iota(jnp.int32, (1024, 1024), 0)
        == lax.broadcasted_iota(jnp.int32, (1024, 1024), 1),
        jnp.float32(1.0), jnp.float32(0.0))

    def body(j, acc):
        b_row = ms_ref[pl.ds(j, 1)].reshape(1, 1024)
        b_col = _to_col(b_row, eye)
        j_idx = j * 1024 + lax.broadcasted_iota(jnp.int32, (1024, 1024), 0)
        ahead = (b_col > a_row) | ((b_col == a_row) & (j_idx > i_idx))
        cnt = jnp.sum(jnp.where(ahead, 1.0, 0.0), axis=0, keepdims=True)
        return acc + cnt

    rank = lax.fori_loop(0, n_chunk, body, jnp.zeros((1, 1024), jnp.float32))
    out_ref[0] = rank


def _rank(ms3):
    return pl.pallas_call(
        _rank_kernel,
        grid=(_N // 1024,),
        in_specs=[pl.BlockSpec((_N // 1024, 1, 1024), lambda i: (0, 0, 0))],
        out_specs=pl.BlockSpec((1, 1, 1024), lambda i: (i, 0, 0)),
        out_shape=jax.ShapeDtypeStruct((_N // 1024, 1, 1024), jnp.float32),
    )(ms3)


def _sc_sort_gather(rank_i, planes5):
    mesh = plsc.VectorSubcoreMesh(core_axis_name="c", subcore_axis_name="s")

    @functools.partial(
        pl.kernel,
        out_type=jax.ShapeDtypeStruct((5 * _NS,), jnp.float32),
        mesh=mesh,
        scratch_types=[
            pltpu.VMEM((_N,), jnp.int32),
            pltpu.VMEM((_NS,), jnp.int32),
            pltpu.VMEM((_N,), jnp.float32),
            pltpu.VMEM((_NS,), jnp.float32),
        ],
        compiler_params=pltpu.CompilerParams(needs_layout_passes=False),
    )
    def run(rank_hbm, planes_hbm, out_hbm, rank_v, order_v, plane_v, sorted_v):
        wid = lax.axis_index("s") * 2 + lax.axis_index("c")

        @pl.when(wid == 0)
        def _():
            pltpu.sync_copy(rank_hbm, rank_v)

            def scat(r, carry):
                idx = rank_v[pl.ds(r * 16, 16)]
                vals = lax.iota(jnp.int32, 16) + r * 16
                plsc.store_scatter(order_v, [idx], vals, mask=idx < _NS)
                return carry

            lax.fori_loop(0, _N // 16, scat, 0)

            for c in range(5):
                pltpu.sync_copy(planes_hbm.at[pl.ds(c * _N, _N)], plane_v)

                def gat(g, carry):
                    idx = order_v[pl.ds(g * 16, 16)]
                    sorted_v[pl.ds(g * 16, 16)] = plsc.load_gather(
                        plane_v, [idx])
                    return carry

                lax.fori_loop(0, _NS // 16, gat, 0)
                pltpu.sync_copy(sorted_v, out_hbm.at[pl.ds(c * _NS, _NS)])

    return run(rank_i, planes5).reshape(5, _NS)


def _nms_kernel(pr_ref, out_ref, sup_ref):
    thr = jnp.float32(_NMS_IOU)
    lane = lax.broadcasted_iota(jnp.int32, (1, _B), 1)
    eye = jnp.where(
        lax.broadcasted_iota(jnp.int32, (_B, _B), 0)
        == lax.broadcasted_iota(jnp.int32, (_B, _B), 1),
        jnp.float32(1.0), jnp.float32(0.0))
    tri = jnp.where(
        lax.broadcasted_iota(jnp.int32, (_B, _B), 0)
        < lax.broadcasted_iota(jnp.int32, (_B, _B), 1),
        jnp.float32(1.0), jnp.float32(0.0))

    def row(c, bi):
        return pr_ref[c, bi]

    def row_dyn(c, bj):
        return pr_ref[c, pl.ds(bj, 1)].reshape(1, _B)

    for bi in range(_NB):
        ms = row(4, bi)
        gidx = bi * _B + lane
        sup0 = jnp.where((ms == _NEG) | (gidx >= _PRE_NMS), 1.0, 0.0)
        sup_ref[bi] = sup0

    for bi in range(_NB):
        bx1, by1, bx2, by2 = (row(c, bi) for c in range(4))
        areas = (bx2 - bx1) * (by2 - by1)
        loc = lane

        def intra(j, sup):
            oh = jnp.where(loc == j, 1.0, 0.0)
            xj1 = jnp.sum(bx1 * oh)
            yj1 = jnp.sum(by1 * oh)
            xj2 = jnp.sum(bx2 * oh)
            yj2 = jnp.sum(by2 * oh)
            aj = jnp.sum(areas * oh)
            supj = jnp.sum(sup * oh)
            xx1 = jnp.maximum(xj1, bx1)
            yy1 = jnp.maximum(yj1, by1)
            xx2 = jnp.minimum(xj2, bx2)
            yy2 = jnp.minimum(yj2, by2)
            inter = jnp.maximum(xx2 - xx1, 0.0) * jnp.maximum(yy2 - yy1, 0.0)
            iou = inter / (aj + areas - inter + 1e-9)
            hit = jnp.where((iou > thr) & (loc > j), 1.0, 0.0)
            return jnp.maximum(sup, hit * (1.0 - supj))

        sup_b = lax.fori_loop(0, _B, intra, sup_ref[bi][...])
        sup_ref[bi] = sup_b

        if bi + 1 < _NB:
            kept_col = _to_col(1.0 - sup_b, eye)
            ix1 = _to_col(bx1, eye)
            iy1 = _to_col(by1, eye)
            ix2 = _to_col(bx2, eye)
            iy2 = _to_col(by2, eye)
            ia = _to_col(areas, eye)

            def suffix(bj, carry):
                sx1 = row_dyn(0, bj)
                sy1 = row_dyn(1, bj)
                sx2 = row_dyn(2, bj)
                sy2 = row_dyn(3, bj)
                sa = (sx2 - sx1) * (sy2 - sy1)
                xx1 = jnp.maximum(ix1, sx1)
                yy1 = jnp.maximum(iy1, sy1)
                xx2 = jnp.minimum(ix2, sx2)
                yy2 = jnp.minimum(iy2, sy2)
                inter = (jnp.maximum(xx2 - xx1, 0.0)
                         * jnp.maximum(yy2 - yy1, 0.0))
                iou = inter / (ia + sa - inter + 1e-9)
                hit = jnp.where(iou > thr, 1.0, 0.0) * kept_col
                contrib = jnp.max(hit, axis=0, keepdims=True)
                old = sup_ref[pl.ds(bj, 1)].reshape(1, _B)
                sup_ref[pl.ds(bj, 1)] = jnp.maximum(old, contrib).reshape(
                    1, 1, _B)
                return carry

            lax.fori_loop(bi + 1, _NB, suffix, 0)

    psel = lax.broadcasted_iota(jnp.int32, (1, _SEL), 1).astype(jnp.float32)
    outs = [jnp.zeros((1, _SEL), jnp.float32) for _ in range(4)]
    nk = jnp.float32(0.0)
    for bi in range(_NB):
        kept = 1.0 - sup_ref[bi][...]
        prefix = lax.dot_general(kept, tri, (((1,), (0,)), ((), ())),
                                 precision=lax.Precision.HIGHEST,
                                 preferred_element_type=jnp.float32)
        pos = nk + prefix
        nk = nk + jnp.sum(kept)
        pos_col = _to_col(pos, eye)
        kept_col = _to_col(kept, eye)
        g2 = jnp.where((pos_col == psel) & (kept_col > 0.5), 1.0, 0.0)
        for c in range(4):
            contrib = lax.dot_general(row(c, bi), g2, (((1,), (0,)), ((), ())),
                                      precision=lax.Precision.HIGHEST,
                                      preferred_element_type=jnp.float32)
            outs[c] = outs[c] + contrib
    oh0 = jnp.where(lane == 0, 1.0, 0.0)
    fill = jnp.where(psel >= nk, 1.0, 0.0)
    for c in range(4):
        x0 = jnp.sum(row(c, 0) * oh0)
        outs[c] = outs[c] + x0 * fill
    out_ref[...] = jnp.concatenate(
        outs + [jnp.zeros((4, _SEL), jnp.float32)], axis=0)


def _nms_select(planes_row):
    return pl.pallas_call(
        _nms_kernel,
        scratch_shapes=[pltpu.VMEM((_NB, 1, _B), jnp.float32)],
        out_shape=jax.ShapeDtypeStruct((8, _SEL), jnp.float32),
    )(planes_row)


def kernel(x, img_size, W_conv1, b_conv1, W_score, b_score, W_loc, b_loc):
    xt = jnp.transpose(x[0], (1, 2, 0))
    xpad = jnp.pad(xt, ((1, 1), (1, 1), (0, 0)))
    xcol = jnp.stack([
        xpad[ky:ky + _H, kx:kx + _W, :].reshape(_P, 512)
        for ky in range(3) for kx in range(3)
    ])
    w9 = jnp.stack([
        jnp.transpose(W_conv1[:, :, ky, kx])
        for ky in range(3) for kx in range(3)
    ])

    facc = _conv_trunk(xcol, w9)

    wh = jnp.pad(
        jnp.concatenate([jnp.transpose(W_loc.reshape(36, 512)),
                         jnp.transpose(W_score.reshape(18, 512))], axis=1),
        ((0, 0), (0, 128 - 54)))
    bh = jnp.pad(jnp.concatenate([b_loc, b_score]), (0, 128 - 54))
    head = _head(facc, b_conv1.reshape(1, 512), wh, bh.reshape(1, 128))

    rpn_locs = head[:, :36].reshape(1, _N, 4)
    rpn_scores = head[:, 36:54].reshape(1, _N, 2)

    anchor = jnp.asarray(_anchor_grid())
    locs_flat = rpn_locs[0]
    scores_flat = rpn_scores[0]
    planes10 = jnp.stack([
        locs_flat[:, 0], locs_flat[:, 1], locs_flat[:, 2], locs_flat[:, 3],
        scores_flat[:, 0], scores_flat[:, 1],
        anchor[:, 0], anchor[:, 1], anchor[:, 2], anchor[:, 3],
    ]).reshape(10, 72, 128)
    img_f = img_size.astype(jnp.float32)

    out5 = _decode(planes10, img_f)
    ms3 = out5[4].reshape(_N // 1024, 1, 1024)
    rank_f = _rank(ms3)
    rank_i = rank_f.reshape(_N).astype(jnp.int32)
    planes5 = out5.reshape(5 * _N)

    sorted5 = _sc_sort_gather(rank_i, planes5)
    planes_row = sorted5.reshape(5, _NB, 1, _B)

    sel = _nms_select(planes_row)
    rois = jnp.transpose(sel[0:4, 0:_POST_NMS])[None]
    roi_indices = jnp.zeros((1, _POST_NMS), jnp.float32)

    return (rpn_locs, rpn_scores, rois, roi_indices,
            anchor[None].astype(jnp.float32))


# speedup vs baseline: 21.0160x; 2.2657x over previous
import functools

import numpy as np
import jax
import jax.numpy as jnp
from jax import lax
from jax.experimental import pallas as pl
from jax.experimental.pallas import tpu as pltpu
from jax.experimental.pallas import tpu_sc as plsc

_RATIOS = (0.5, 1.0, 2.0)
_SCALES = (8, 16, 32)
_FEAT_STRIDE = 16
_BASE_SIZE = 16
_NMS_IOU = 0.7
_PRE_NMS = 3000
_POST_NMS = 300
_MIN_SIZE = 16.0

_H = 32
_W = 32
_P = _H * _W
_A = 9
_N = _P * _A
_NS = 3072
_B = 128
_NB = _NS // _B
_SEL = 384

_NEG = -1.0e38


def _anchor_grid():
    ab = np.zeros((9, 4), dtype=np.float32)
    for i, r in enumerate(_RATIOS):
        for j, s in enumerate(_SCALES):
            h = _BASE_SIZE * s * np.sqrt(r)
            w = _BASE_SIZE * s * np.sqrt(1.0 / r)
            k = i * 3 + j
            ab[k, 0] = -w / 2.0
            ab[k, 1] = -h / 2.0
            ab[k, 2] = w / 2.0
            ab[k, 3] = h / 2.0
    sx = np.arange(0, _W * _FEAT_STRIDE, _FEAT_STRIDE)
    sy = np.arange(0, _H * _FEAT_STRIDE, _FEAT_STRIDE)
    mx, my = np.meshgrid(sx, sy)
    shift = np.stack([mx.ravel(), my.ravel(), mx.ravel(), my.ravel()],
                     axis=1).astype(np.float32)
    return (ab[None, :, :] + shift[:, None, :]).reshape(-1, 4)


def _dot(a, b, prec):
    return lax.dot_general(a, b, (((1,), (0,)), ((), ())), precision=prec,
                           preferred_element_type=jnp.float32)


def _to_col(v_row, eye):
    return lax.dot_general(eye, v_row, (((1,), (1,)), ((), ())),
                           precision=lax.Precision.HIGHEST,
                           preferred_element_type=jnp.float32)


def _conv_kernel(x_ref, w_ref, out_ref):
    k = pl.program_id(0)
    part = _dot(x_ref[0], w_ref[0], lax.Precision.DEFAULT)

    @pl.when(k == 0)
    def _():
        out_ref[...] = part

    @pl.when(k > 0)
    def _():
        out_ref[...] = out_ref[...] + part


def _conv_trunk(xcol, w9):
    return pl.pallas_call(
        _conv_kernel,
        grid=(9,),
        in_specs=[
            pl.BlockSpec((1, _P, 512), lambda k: (k, 0, 0)),
            pl.BlockSpec((1, 512, 512), lambda k: (k, 0, 0)),
        ],
        out_specs=pl.BlockSpec((_P, 512), lambda k: (0, 0)),
        out_shape=jax.ShapeDtypeStruct((_P, 512), jnp.float32),
    )(xcol, w9)


def _head_kernel(f_ref, bc_ref, wh_ref, bh_ref, out_ref):
    f = jnp.maximum(f_ref[...] + bc_ref[...], 0.0)
    out_ref[...] = _dot(f, wh_ref[...], lax.Precision.DEFAULT) + bh_ref[...]


def _head(facc, bc, wh, bh):
    return pl.pallas_call(
        _head_kernel,
        out_shape=jax.ShapeDtypeStruct((_P, 128), jnp.float32),
    )(facc, bc, wh, bh)


def _decode_kernel(p_ref, img_ref, out_ref):
    dx, dy, dw, dh = p_ref[0], p_ref[1], p_ref[2], p_ref[3]
    s0, s1 = p_ref[4], p_ref[5]
    ax1, ay1, ax2, ay2 = p_ref[6], p_ref[7], p_ref[8], p_ref[9]
    imh = img_ref[0]
    imw = img_ref[1]

    sw = ax2 - ax1
    sh = ay2 - ay1
    cx0 = ax1 + 0.5 * sw
    cy0 = ay1 + 0.5 * sh
    cx = dx * sw + cx0
    cy = dy * sh + cy0
    ww = jnp.exp(dw) * sw
    hh = jnp.exp(dh) * sh
    x1 = jnp.clip(cx - 0.5 * ww, 0.0, imw)
    y1 = jnp.clip(cy - 0.5 * hh, 0.0, imh)
    x2 = jnp.clip(cx + 0.5 * ww, 0.0, imw)
    y2 = jnp.clip(cy + 0.5 * hh, 0.0, imh)
    valid = (x2 - x1 >= _MIN_SIZE) & (y2 - y1 >= _MIN_SIZE)

    m = jnp.maximum(s0, s1)
    e0 = jnp.exp(s0 - m)
    e1 = jnp.exp(s1 - m)
    fg = e1 / (e0 + e1)
    ms = jnp.where(valid, fg, _NEG)

    out_ref[0] = x1
    out_ref[1] = y1
    out_ref[2] = x2
    out_ref[3] = y2
    out_ref[4] = ms


def _decode(planes10, img_f):
    return pl.pallas_call(
        _decode_kernel,
        in_specs=[
            pl.BlockSpec(memory_space=pltpu.VMEM),
            pl.BlockSpec(memory_space=pltpu.SMEM),
        ],
        out_shape=jax.ShapeDtypeStruct((5, 72, 128), jnp.float32),
    )(planes10, img_f)


def _rank_kernel(ms_ref, out_ref):
    i = pl.program_id(0)
    n_chunk = _N // 1024
    a_row = ms_ref[pl.ds(i, 1)].reshape(1, 1024)
    i_idx = i * 1024 + lax.broadcasted_iota(jnp.int32, (1024, 1024), 1)
    eye = jnp.where(
        lax.broadcasted_iota(jnp.int32, (1024, 1024), 0)
        == lax.broadcasted_iota(jnp.int32, (1024, 1024), 1),
        jnp.float32(1.0), jnp.float32(0.0))

    def body(j, acc):
        b_row = ms_ref[pl.ds(j, 1)].reshape(1, 1024)
        b_col = _to_col(b_row, eye)
        j_idx = j * 1024 + lax.broadcasted_iota(jnp.int32, (1024, 1024), 0)
        ahead = (b_col > a_row) | ((b_col == a_row) & (j_idx > i_idx))
        cnt = jnp.sum(jnp.where(ahead, 1.0, 0.0), axis=0, keepdims=True)
        return acc + cnt

    rank = lax.fori_loop(0, n_chunk, body, jnp.zeros((1, 1024), jnp.float32))
    out_ref[0] = rank


def _rank(ms3):
    return pl.pallas_call(
        _rank_kernel,
        grid=(_N // 1024,),
        in_specs=[pl.BlockSpec((_N // 1024, 1, 1024), lambda i: (0, 0, 0))],
        out_specs=pl.BlockSpec((1, 1, 1024), lambda i: (i, 0, 0)),
        out_shape=jax.ShapeDtypeStruct((_N // 1024, 1, 1024), jnp.float32),
    )(ms3)


def _sc_sort_gather(rank_i, planes5):
    mesh = plsc.VectorSubcoreMesh(core_axis_name="c", subcore_axis_name="s")

    @functools.partial(
        pl.kernel,
        out_type=jax.ShapeDtypeStruct((5 * _NS,), jnp.float32),
        mesh=mesh,
        scratch_types=[
            pltpu.VMEM((_N,), jnp.int32),
            pltpu.VMEM((_NS,), jnp.int32),
            pltpu.VMEM((_N,), jnp.float32),
            pltpu.VMEM((_NS,), jnp.float32),
        ],
        compiler_params=pltpu.CompilerParams(needs_layout_passes=False),
    )
    def run(rank_hbm, planes_hbm, out_hbm, rank_v, order_v, plane_v, sorted_v):
        wid = lax.axis_index("s") * 2 + lax.axis_index("c")

        @pl.when(wid == 0)
        def _():
            pltpu.sync_copy(rank_hbm, rank_v)

            def scat(r, carry):
                idx = rank_v[pl.ds(r * 16, 16)]
                vals = lax.iota(jnp.int32, 16) + r * 16
                plsc.store_scatter(order_v, [idx], vals, mask=idx < _NS)
                return carry

            lax.fori_loop(0, _N // 16, scat, 0)

            for c in range(5):
                pltpu.sync_copy(planes_hbm.at[pl.ds(c * _N, _N)], plane_v)

                def gat(g, carry):
                    idx = order_v[pl.ds(g * 16, 16)]
                    sorted_v[pl.ds(g * 16, 16)] = plsc.load_gather(
                        plane_v, [idx])
                    return carry

                lax.fori_loop(0, _NS // 16, gat, 0)
                pltpu.sync_copy(sorted_v, out_hbm.at[pl.ds(c * _NS, _NS)])

    return run(rank_i, planes5).reshape(5, _NS)


def _nms_kernel(pr_ref, out_ref, sup_ref):
    thr = jnp.float32(_NMS_IOU)
    lane = lax.broadcasted_iota(jnp.int32, (1, _B), 1)
    lane_w = lax.broadcasted_iota(jnp.int32, (1, _NS), 1)
    eye = jnp.where(
        lax.broadcasted_iota(jnp.int32, (_B, _B), 0)
        == lax.broadcasted_iota(jnp.int32, (_B, _B), 1),
        jnp.float32(1.0), jnp.float32(0.0))
    tri = jnp.where(
        lax.broadcasted_iota(jnp.int32, (_B, _B), 0)
        < lax.broadcasted_iota(jnp.int32, (_B, _B), 1),
        jnp.float32(1.0), jnp.float32(0.0))

    def row(c, bi):
        return pr_ref[c, 0, pl.ds(bi * _B, _B)].reshape(1, _B)

    def wide(c):
        return pr_ref[c]

    sup_ref[0] = jnp.where((wide(4) == _NEG) | (lane_w >= _PRE_NMS), 1.0, 0.0)

    wx1, wy1, wx2, wy2 = wide(0), wide(1), wide(2), wide(3)
    wa = (wx2 - wx1) * (wy2 - wy1)

    for bi in range(_NB):
        bx1, by1, bx2, by2 = (row(c, bi) for c in range(4))
        areas = (bx2 - bx1) * (by2 - by1)
        x1c = _to_col(bx1, eye)
        y1c = _to_col(by1, eye)
        x2c = _to_col(bx2, eye)
        y2c = _to_col(by2, eye)
        ac = _to_col(areas, eye)

        xx1 = jnp.maximum(x1c, bx1)
        yy1 = jnp.maximum(y1c, by1)
        xx2 = jnp.minimum(x2c, bx2)
        yy2 = jnp.minimum(y2c, by2)
        inter = jnp.maximum(xx2 - xx1, 0.0) * jnp.maximum(yy2 - yy1, 0.0)
        iou = inter / (ac + areas - inter + 1e-9)
        jlti = (lax.broadcasted_iota(jnp.int32, (_B, _B), 0)
                < lax.broadcasted_iota(jnp.int32, (_B, _B), 1))
        omat = jnp.where((iou > thr) & jlti, 1.0, 0.0)

        v = 1.0 - sup_ref[0, 0, pl.ds(bi * _B, _B)].reshape(1, _B)

        def fstep(k):
            kcol = _to_col(k, eye)
            m = jnp.max(omat * kcol, axis=0, keepdims=True)
            return v * (1.0 - m)

        def cond(c):
            return jnp.any(c[0] != c[1])

        def body(c):
            kn = c[1]
            return (kn, fstep(kn))

        k0 = v
        kfix = lax.while_loop(cond, body, (k0, fstep(k0)))[1]
        sup_ref[0, 0, pl.ds(bi * _B, _B)] = (1.0 - kfix).reshape(_B)

        if bi + 1 < _NB:
            kept_col = _to_col(kfix, eye)
            xx1w = jnp.maximum(x1c, wx1)
            yy1w = jnp.maximum(y1c, wy1)
            xx2w = jnp.minimum(x2c, wx2)
            yy2w = jnp.minimum(y2c, wy2)
            interw = (jnp.maximum(xx2w - xx1w, 0.0)
                      * jnp.maximum(yy2w - yy1w, 0.0))
            iouw = interw / (ac + wa - interw + 1e-9)
            hit = jnp.where(iouw > thr, 1.0, 0.0) * kept_col
            contrib = jnp.max(hit, axis=0, keepdims=True)
            later = jnp.where(lane_w >= (bi + 1) * _B, 1.0, 0.0)
            sup_ref[0] = jnp.maximum(sup_ref[0][...], contrib * later)

    psel = lax.broadcasted_iota(jnp.int32, (1, _SEL), 1).astype(jnp.float32)
    outs = [jnp.zeros((1, _SEL), jnp.float32) for _ in range(4)]
    nk = jnp.float32(0.0)
    for bi in range(_NB):
        kept = 1.0 - sup_ref[0, 0, pl.ds(bi * _B, _B)].reshape(1, _B)
        prefix = lax.dot_general(kept, tri, (((1,), (0,)), ((), ())),
                                 precision=lax.Precision.HIGHEST,
                                 preferred_element_type=jnp.float32)
        pos = nk + prefix
        nk = nk + jnp.sum(kept)
        pos_col = _to_col(pos, eye)
        kept_col = _to_col(kept, eye)
        g2 = jnp.where((pos_col == psel) & (kept_col > 0.5), 1.0, 0.0)
        for c in range(4):
            contrib = lax.dot_general(row(c, bi), g2, (((1,), (0,)), ((), ())),
                                      precision=lax.Precision.HIGHEST,
                                      preferred_element_type=jnp.float32)
            outs[c] = outs[c] + contrib
    oh0 = jnp.where(lane == 0, 1.0, 0.0)
    fill = jnp.where(psel >= nk, 1.0, 0.0)
    for c in range(4):
        x0 = jnp.sum(row(c, 0) * oh0)
        outs[c] = outs[c] + x0 * fill
    out_ref[...] = jnp.concatenate(
        outs + [jnp.zeros((4, _SEL), jnp.float32)], axis=0)


def _nms_select(planes_full):
    return pl.pallas_call(
        _nms_kernel,
        scratch_shapes=[pltpu.VMEM((1, 1, _NS), jnp.float32)],
        out_shape=jax.ShapeDtypeStruct((8, _SEL), jnp.float32),
    )(planes_full)


def kernel(x, img_size, W_conv1, b_conv1, W_score, b_score, W_loc, b_loc):
    xt = jnp.transpose(x[0], (1, 2, 0))
    xpad = jnp.pad(xt, ((1, 1), (1, 1), (0, 0)))
    xcol = jnp.stack([
        xpad[ky:ky + _H, kx:kx + _W, :].reshape(_P, 512)
        for ky in range(3) for kx in range(3)
    ])
    w9 = jnp.stack([
        jnp.transpose(W_conv1[:, :, ky, kx])
        for ky in range(3) for kx in range(3)
    ])

    facc = _conv_trunk(xcol, w9)

    wh = jnp.pad(
        jnp.concatenate([jnp.transpose(W_loc.reshape(36, 512)),
                         jnp.transpose(W_score.reshape(18, 512))], axis=1),
        ((0, 0), (0, 128 - 54)))
    bh = jnp.pad(jnp.concatenate([b_loc, b_score]), (0, 128 - 54))
    head = _head(facc, b_conv1.reshape(1, 512), wh, bh.reshape(1, 128))

    rpn_locs = head[:, :36].reshape(1, _N, 4)
    rpn_scores = head[:, 36:54].reshape(1, _N, 2)

    anchor = jnp.asarray(_anchor_grid())
    locs_flat = rpn_locs[0]
    scores_flat = rpn_scores[0]
    planes10 = jnp.stack([
        locs_flat[:, 0], locs_flat[:, 1], locs_flat[:, 2], locs_flat[:, 3],
        scores_flat[:, 0], scores_flat[:, 1],
        anchor[:, 0], anchor[:, 1], anchor[:, 2], anchor[:, 3],
    ]).reshape(10, 72, 128)
    img_f = img_size.astype(jnp.float32)

    out5 = _decode(planes10, img_f)
    ms3 = out5[4].reshape(_N // 1024, 1, 1024)
    rank_f = _rank(ms3)
    rank_i = rank_f.reshape(_N).astype(jnp.int32)
    planes5 = out5.reshape(5 * _N)

    sorted5 = _sc_sort_gather(rank_i, planes5)
    planes_full = sorted5.reshape(5, 1, _NS)

    sel = _nms_select(planes_full)
    rois = jnp.transpose(sel[0:4, 0:_POST_NMS])[None]
    roi_indices = jnp.zeros((1, _POST_NMS), jnp.float32)

    return (rpn_locs, rpn_scores, rois, roi_indices,
            anchor[None].astype(jnp.float32))


# speedup vs baseline: 22.1021x; 1.0517x over previous
import functools

import numpy as np
import jax
import jax.numpy as jnp
from jax import lax
from jax.experimental import pallas as pl
from jax.experimental.pallas import tpu as pltpu
from jax.experimental.pallas import tpu_sc as plsc

_RATIOS = (0.5, 1.0, 2.0)
_SCALES = (8, 16, 32)
_FEAT_STRIDE = 16
_BASE_SIZE = 16
_NMS_IOU = 0.7
_PRE_NMS = 3000
_POST_NMS = 300
_MIN_SIZE = 16.0

_H = 32
_W = 32
_P = _H * _W
_A = 9
_N = _P * _A
_NS = 3072
_B = 128
_NB = _NS // _B
_SEL = 384

_NEG = -1.0e38


def _anchor_grid():
    ab = np.zeros((9, 4), dtype=np.float32)
    for i, r in enumerate(_RATIOS):
        for j, s in enumerate(_SCALES):
            h = _BASE_SIZE * s * np.sqrt(r)
            w = _BASE_SIZE * s * np.sqrt(1.0 / r)
            k = i * 3 + j
            ab[k, 0] = -w / 2.0
            ab[k, 1] = -h / 2.0
            ab[k, 2] = w / 2.0
            ab[k, 3] = h / 2.0
    sx = np.arange(0, _W * _FEAT_STRIDE, _FEAT_STRIDE)
    sy = np.arange(0, _H * _FEAT_STRIDE, _FEAT_STRIDE)
    mx, my = np.meshgrid(sx, sy)
    shift = np.stack([mx.ravel(), my.ravel(), mx.ravel(), my.ravel()],
                     axis=1).astype(np.float32)
    return (ab[None, :, :] + shift[:, None, :]).reshape(-1, 4)


def _dot(a, b, prec):
    return lax.dot_general(a, b, (((1,), (0,)), ((), ())), precision=prec,
                           preferred_element_type=jnp.float32)


def _to_col(v_row, eye):
    return lax.dot_general(eye, v_row, (((1,), (1,)), ((), ())),
                           precision=lax.Precision.HIGHEST,
                           preferred_element_type=jnp.float32)


def _trunk_kernel(x_ref, w_ref, bc_ref, wh_ref, bh_ref, out_ref, acc_ref):
    k = pl.program_id(0)
    part = _dot(x_ref[0], w_ref[0], lax.Precision.DEFAULT)

    @pl.when(k == 0)
    def _():
        acc_ref[...] = part

    @pl.when(k > 0)
    def _():
        acc_ref[...] = acc_ref[...] + part

    @pl.when(k == 8)
    def _():
        f = jnp.maximum(acc_ref[...] + bc_ref[...], 0.0)
        out_ref[...] = (_dot(f, wh_ref[...], lax.Precision.DEFAULT)
                        + bh_ref[...])


def _trunk(xcol, w9, bc, wh, bh):
    return pl.pallas_call(
        _trunk_kernel,
        grid=(9,),
        in_specs=[
            pl.BlockSpec((1, _P, 512), lambda k: (k, 0, 0)),
            pl.BlockSpec((1, 512, 512), lambda k: (k, 0, 0)),
            pl.BlockSpec((1, 512), lambda k: (0, 0)),
            pl.BlockSpec((512, 128), lambda k: (0, 0)),
            pl.BlockSpec((1, 128), lambda k: (0, 0)),
        ],
        out_specs=pl.BlockSpec((_P, 128), lambda k: (0, 0)),
        out_shape=jax.ShapeDtypeStruct((_P, 128), jnp.float32),
        scratch_shapes=[pltpu.VMEM((_P, 512), jnp.float32)],
    )(xcol, w9, bc, wh, bh)


def _decode_kernel(p_ref, img_ref, out_ref):
    dx, dy, dw, dh = p_ref[0], p_ref[1], p_ref[2], p_ref[3]
    s0, s1 = p_ref[4], p_ref[5]
    ax1, ay1, ax2, ay2 = p_ref[6], p_ref[7], p_ref[8], p_ref[9]
    imh = img_ref[0]
    imw = img_ref[1]

    sw = ax2 - ax1
    sh = ay2 - ay1
    cx0 = ax1 + 0.5 * sw
    cy0 = ay1 + 0.5 * sh
    cx = dx * sw + cx0
    cy = dy * sh + cy0
    ww = jnp.exp(dw) * sw
    hh = jnp.exp(dh) * sh
    x1 = jnp.clip(cx - 0.5 * ww, 0.0, imw)
    y1 = jnp.clip(cy - 0.5 * hh, 0.0, imh)
    x2 = jnp.clip(cx + 0.5 * ww, 0.0, imw)
    y2 = jnp.clip(cy + 0.5 * hh, 0.0, imh)
    valid = (x2 - x1 >= _MIN_SIZE) & (y2 - y1 >= _MIN_SIZE)

    m = jnp.maximum(s0, s1)
    e0 = jnp.exp(s0 - m)
    e1 = jnp.exp(s1 - m)
    fg = e1 / (e0 + e1)
    ms = jnp.where(valid, fg, _NEG)

    out_ref[0] = x1
    out_ref[1] = y1
    out_ref[2] = x2
    out_ref[3] = y2
    out_ref[4] = ms


def _decode(planes10, img_f):
    return pl.pallas_call(
        _decode_kernel,
        in_specs=[
            pl.BlockSpec(memory_space=pltpu.VMEM),
            pl.BlockSpec(memory_space=pltpu.SMEM),
        ],
        out_shape=jax.ShapeDtypeStruct((5, 72, 128), jnp.float32),
    )(planes10, img_f)


def _rank_kernel(ms_ref, out_ref):
    i = pl.program_id(0)
    n_chunk = _N // 1024
    eye = jnp.where(
        lax.broadcasted_iota(jnp.int32, (1024, 1024), 0)
        == lax.broadcasted_iota(jnp.int32, (1024, 1024), 1),
        jnp.float32(1.0), jnp.float32(0.0))
    a_row = ms_ref[pl.ds(i, 1)].reshape(1, 1024)
    a_col = _to_col(a_row, eye)
    dmat = (lax.broadcasted_iota(jnp.int32, (1024, 1024), 1)
            - lax.broadcasted_iota(jnp.int32, (1024, 1024), 0))

    def body(j, acc):
        b_row = ms_ref[pl.ds(j, 1)].reshape(1, 1024)
        c = (i - j) * 1024
        ahead = (b_row > a_col) | ((b_row == a_col) & (dmat > c))
        cnt = jnp.sum(jnp.where(ahead, 1.0, 0.0), axis=1, keepdims=True)
        return acc + cnt

    rank_col = lax.fori_loop(0, n_chunk, body,
                             jnp.zeros((1024, 1), jnp.float32))
    out_ref[0] = lax.dot_general(rank_col, eye, (((0,), (0,)), ((), ())),
                                 precision=lax.Precision.HIGHEST,
                                 preferred_element_type=jnp.float32)


def _rank(ms3):
    return pl.pallas_call(
        _rank_kernel,
        grid=(_N // 1024,),
        in_specs=[pl.BlockSpec((_N // 1024, 1, 1024), lambda i: (0, 0, 0))],
        out_specs=pl.BlockSpec((1, 1, 1024), lambda i: (i, 0, 0)),
        out_shape=jax.ShapeDtypeStruct((_N // 1024, 1, 1024), jnp.float32),
    )(ms3)


def _sc_sort_gather(rank_i, planes5):
    mesh = plsc.VectorSubcoreMesh(core_axis_name="c", subcore_axis_name="s")

    @functools.partial(
        pl.kernel,
        out_type=jax.ShapeDtypeStruct((5 * _NS,), jnp.float32),
        mesh=mesh,
        scratch_types=[
            pltpu.VMEM((_N,), jnp.int32),
            pltpu.VMEM((_NS,), jnp.int32),
            pltpu.VMEM((_N,), jnp.float32),
            pltpu.VMEM((_NS,), jnp.float32),
        ],
        compiler_params=pltpu.CompilerParams(needs_layout_passes=False),
    )
    def run(rank_hbm, planes_hbm, out_hbm, rank_v, order_v, plane_v, sorted_v):
        wid = lax.axis_index("s") * 2 + lax.axis_index("c")

        @pl.when(wid == 0)
        def _():
            pltpu.sync_copy(rank_hbm, rank_v)

            def scat(r, carry):
                idx = rank_v[pl.ds(r * 16, 16)]
                vals = lax.iota(jnp.int32, 16) + r * 16
                plsc.store_scatter(order_v, [idx], vals, mask=idx < _NS)
                return carry

            lax.fori_loop(0, _N // 16, scat, 0)

            for c in range(5):
                pltpu.sync_copy(planes_hbm.at[pl.ds(c * _N, _N)], plane_v)

                def gat(g, carry):
                    idx = order_v[pl.ds(g * 16, 16)]
                    sorted_v[pl.ds(g * 16, 16)] = plsc.load_gather(
                        plane_v, [idx])
                    return carry

                lax.fori_loop(0, _NS // 16, gat, 0)
                pltpu.sync_copy(sorted_v, out_hbm.at[pl.ds(c * _NS, _NS)])

    return run(rank_i, planes5).reshape(5, _NS)


def _nms_kernel(pr_ref, out_ref, sup_ref):
    thr = jnp.float32(_NMS_IOU)
    lane = lax.broadcasted_iota(jnp.int32, (1, _B), 1)
    lane_w = lax.broadcasted_iota(jnp.int32, (1, _NS), 1)
    eye = jnp.where(
        lax.broadcasted_iota(jnp.int32, (_B, _B), 0)
        == lax.broadcasted_iota(jnp.int32, (_B, _B), 1),
        jnp.float32(1.0), jnp.float32(0.0))
    tri = jnp.where(
        lax.broadcasted_iota(jnp.int32, (_B, _B), 0)
        < lax.broadcasted_iota(jnp.int32, (_B, _B), 1),
        jnp.float32(1.0), jnp.float32(0.0))

    def row(c, bi):
        return pr_ref[c, 0, pl.ds(bi * _B, _B)].reshape(1, _B)

    def wide(c):
        return pr_ref[c]

    sup_ref[0] = jnp.where((wide(4) == _NEG) | (lane_w >= _PRE_NMS), 1.0, 0.0)

    wx1, wy1, wx2, wy2 = wide(0), wide(1), wide(2), wide(3)
    wa = (wx2 - wx1) * (wy2 - wy1)

    for bi in range(_NB):
        bx1, by1, bx2, by2 = (row(c, bi) for c in range(4))
        areas = (bx2 - bx1) * (by2 - by1)
        x1c = _to_col(bx1, eye)
        y1c = _to_col(by1, eye)
        x2c = _to_col(bx2, eye)
        y2c = _to_col(by2, eye)
        ac = _to_col(areas, eye)

        xx1 = jnp.maximum(x1c, bx1)
        yy1 = jnp.maximum(y1c, by1)
        xx2 = jnp.minimum(x2c, bx2)
        yy2 = jnp.minimum(y2c, by2)
        inter = jnp.maximum(xx2 - xx1, 0.0) * jnp.maximum(yy2 - yy1, 0.0)
        iou = inter / (ac + areas - inter + 1e-9)
        jlti = (lax.broadcasted_iota(jnp.int32, (_B, _B), 0)
                < lax.broadcasted_iota(jnp.int32, (_B, _B), 1))
        omat = jnp.where((iou > thr) & jlti, 1.0, 0.0)

        v = 1.0 - sup_ref[0, 0, pl.ds(bi * _B, _B)].reshape(1, _B)

        def fstep(k):
            kcol = _to_col(k, eye)
            m = jnp.max(omat * kcol, axis=0, keepdims=True)
            return v * (1.0 - m)

        def cond(c):
            return jnp.any(c[0] != c[1])

        def body(c):
            kn = c[1]
            return (kn, fstep(kn))

        k0 = v
        kfix = lax.while_loop(cond, body, (k0, fstep(k0)))[1]
        sup_ref[0, 0, pl.ds(bi * _B, _B)] = (1.0 - kfix).reshape(_B)

        if bi + 1 < _NB:
            kept_col = _to_col(kfix, eye)
            xx1w = jnp.maximum(x1c, wx1)
            yy1w = jnp.maximum(y1c, wy1)
            xx2w = jnp.minimum(x2c, wx2)
            yy2w = jnp.minimum(y2c, wy2)
            interw = (jnp.maximum(xx2w - xx1w, 0.0)
                      * jnp.maximum(yy2w - yy1w, 0.0))
            iouw = interw / (ac + wa - interw + 1e-9)
            hit = jnp.where(iouw > thr, 1.0, 0.0) * kept_col
            contrib = jnp.max(hit, axis=0, keepdims=True)
            later = jnp.where(lane_w >= (bi + 1) * _B, 1.0, 0.0)
            sup_ref[0] = jnp.maximum(sup_ref[0][...], contrib * later)

    psel = lax.broadcasted_iota(jnp.int32, (1, _SEL), 1).astype(jnp.float32)
    outs = [jnp.zeros((1, _SEL), jnp.float32) for _ in range(4)]
    nk = jnp.float32(0.0)
    for bi in range(_NB):
        kept = 1.0 - sup_ref[0, 0, pl.ds(bi * _B, _B)].reshape(1, _B)
        prefix = lax.dot_general(kept, tri, (((1,), (0,)), ((), ())),
                                 precision=lax.Precision.HIGHEST,
                                 preferred_element_type=jnp.float32)
        pos = nk + prefix
        nk = nk + jnp.sum(kept)
        pos_col = _to_col(pos, eye)
        kept_col = _to_col(kept, eye)
        g2 = jnp.where((pos_col == psel) & (kept_col > 0.5), 1.0, 0.0)
        for c in range(4):
            contrib = lax.dot_general(row(c, bi), g2, (((1,), (0,)), ((), ())),
                                      precision=lax.Precision.HIGHEST,
                                      preferred_element_type=jnp.float32)
            outs[c] = outs[c] + contrib
    oh0 = jnp.where(lane == 0, 1.0, 0.0)
    fill = jnp.where(psel >= nk, 1.0, 0.0)
    for c in range(4):
        x0 = jnp.sum(row(c, 0) * oh0)
        outs[c] = outs[c] + x0 * fill
    out_ref[...] = jnp.concatenate(
        outs + [jnp.zeros((4, _SEL), jnp.float32)], axis=0)


def _nms_select(planes_full):
    return pl.pallas_call(
        _nms_kernel,
        scratch_shapes=[pltpu.VMEM((1, 1, _NS), jnp.float32)],
        out_shape=jax.ShapeDtypeStruct((8, _SEL), jnp.float32),
    )(planes_full)


def kernel(x, img_size, W_conv1, b_conv1, W_score, b_score, W_loc, b_loc):
    xt = jnp.transpose(x[0], (1, 2, 0))
    xpad = jnp.pad(xt, ((1, 1), (1, 1), (0, 0)))
    xcol = jnp.stack([
        xpad[ky:ky + _H, kx:kx + _W, :].reshape(_P, 512)
        for ky in range(3) for kx in range(3)
    ])
    w9 = jnp.stack([
        jnp.transpose(W_conv1[:, :, ky, kx])
        for ky in range(3) for kx in range(3)
    ])

    wh = jnp.pad(
        jnp.concatenate([jnp.transpose(W_loc.reshape(36, 512)),
                         jnp.transpose(W_score.reshape(18, 512))], axis=1),
        ((0, 0), (0, 128 - 54)))
    bh = jnp.pad(jnp.concatenate([b_loc, b_score]), (0, 128 - 54))
    head = _trunk(xcol, w9, b_conv1.reshape(1, 512), wh, bh.reshape(1, 128))

    rpn_locs = head[:, :36].reshape(1, _N, 4)
    rpn_scores = head[:, 36:54].reshape(1, _N, 2)

    anchor = jnp.asarray(_anchor_grid())
    locs_flat = rpn_locs[0]
    scores_flat = rpn_scores[0]
    planes10 = jnp.stack([
        locs_flat[:, 0], locs_flat[:, 1], locs_flat[:, 2], locs_flat[:, 3],
        scores_flat[:, 0], scores_flat[:, 1],
        anchor[:, 0], anchor[:, 1], anchor[:, 2], anchor[:, 3],
    ]).reshape(10, 72, 128)
    img_f = img_size.astype(jnp.float32)

    out5 = _decode(planes10, img_f)
    ms3 = out5[4].reshape(_N // 1024, 1, 1024)
    rank_f = _rank(ms3)
    rank_i = rank_f.reshape(_N).astype(jnp.int32)
    planes5 = out5.reshape(5 * _N)

    sorted5 = _sc_sort_gather(rank_i, planes5)
    planes_full = sorted5.reshape(5, 1, _NS)

    sel = _nms_select(planes_full)
    rois = jnp.transpose(sel[0:4, 0:_POST_NMS])[None]
    roi_indices = jnp.zeros((1, _POST_NMS), jnp.float32)

    return (rpn_locs, rpn_scores, rois, roi_indices,
            anchor[None].astype(jnp.float32))
